# Initial kernel scaffold; baseline (speedup 1.0000x reference)
import functools

import jax
import jax.numpy as jnp
from jax import lax
from jax.experimental import pallas as pl
from jax.experimental.pallas import tpu as pltpu
from jax.experimental.pallas import tpu_sc as plsc

_N = 10000
_E = 320000
_H = 128
_G = 64

_K = 80
_R = _E // _K
_NW = 32
_RW = _R // _NW
_TN = _N // 16
_DW = 8

_sc_mesh = plsc.VectorSubcoreMesh(core_axis_name="c", subcore_axis_name="s")


@functools.partial(
    pl.kernel,
    out_type=jax.ShapeDtypeStruct((2, _N, _DW), jnp.float32),
    mesh=_sc_mesh,
    scratch_types=[
        pltpu.VMEM((_RW, _K), jnp.int32),
        pltpu.VMEM((_K, _DW), jnp.float32),
        pltpu.VMEM((125, _DW), jnp.float32),
        pltpu.VMEM_SHARED((_N, _DW), jnp.float32),
    ],
)
def _deg(dst2d, zeros_hbm, ones_hbm, out, didx, ones_v, wbuf, acc):
    c = lax.axis_index("c")
    s = lax.axis_index("s")
    wid = s * 2 + c
    pltpu.sync_copy(zeros_hbm, wbuf)
    for i in range(5):
        pltpu.sync_copy(wbuf, acc.at[pl.ds(s * _TN + i * 125, 125)])
    pltpu.sync_copy(ones_hbm, ones_v)
    pltpu.sync_copy(dst2d.at[pl.ds(wid * _RW, _RW)], didx)
    plsc.subcore_barrier()

    def body(j, carry):
        pltpu.sync_copy(ones_v, acc.at[didx.at[j]], add=True)
        return carry

    lax.fori_loop(0, _RW, body, 0)
    plsc.subcore_barrier()
    for i in range(5):
        pltpu.sync_copy(acc.at[pl.ds(s * _TN + i * 125, 125)], wbuf)
        pltpu.sync_copy(wbuf, out.at[c, pl.ds(s * _TN + i * 125, 125)])


@functools.partial(
    pl.kernel,
    out_type=jax.ShapeDtypeStruct((2, _N, _H), jnp.float32),
    mesh=_sc_mesh,
    scratch_types=[
        pltpu.VMEM((_RW, _K), jnp.int32),
        pltpu.VMEM((_RW, _K), jnp.int32),
        pltpu.VMEM((_K, _H), jnp.float32),
        pltpu.VMEM((125, _H), jnp.float32),
        pltpu.VMEM_SHARED((_N, _H), jnp.float32),
        pltpu.SemaphoreType.DMA,
    ],
)
def _agg(y, src2d, dst2d, zeros_hbm, out, sidx, didx, buf, wbuf, acc, sem):
    c = lax.axis_index("c")
    s = lax.axis_index("s")
    wid = s * 2 + c
    pltpu.sync_copy(zeros_hbm, wbuf)
    for i in range(5):
        pltpu.sync_copy(wbuf, acc.at[pl.ds(s * _TN + i * 125, 125)])
    pltpu.sync_copy(src2d.at[pl.ds(wid * _RW, _RW)], sidx)
    pltpu.sync_copy(dst2d.at[pl.ds(wid * _RW, _RW)], didx)
    plsc.subcore_barrier()

    def body(j, carry):
        pltpu.async_copy(y.at[sidx.at[j]], buf, sem).wait()
        pltpu.sync_copy(buf, acc.at[didx.at[j]], add=True)
        return carry

    lax.fori_loop(0, _RW, body, 0)
    plsc.subcore_barrier()
    for i in range(5):
        pltpu.sync_copy(acc.at[pl.ds(s * _TN + i * 125, 125)], wbuf)
        pltpu.sync_copy(wbuf, out.at[c, pl.ds(s * _TN + i * 125, 125)])


def _tc_scale(x_ref, w_ref, degp_ref, y_ref, dinv_ref):
    deg = degp_ref[0] + degp_ref[1] + 1.0
    dinv = lax.rsqrt(jnp.maximum(deg, 1.0))
    xw = jnp.dot(x_ref[...], w_ref[...], preferred_element_type=jnp.float32)
    y_ref[...] = xw * dinv[:, 0:1]
    dinv_ref[...] = dinv


def _tc_mid(accp_ref, y_ref, dinv_ref, b_ref, w_ref, y2_ref):
    acc = accp_ref[0] + accp_ref[1] + y_ref[...]
    h = jnp.maximum(acc * dinv_ref[:, 0:1] + b_ref[...], 0.0)
    xw = jnp.dot(h, w_ref[...], preferred_element_type=jnp.float32)
    y2_ref[...] = xw * dinv_ref[:, 0:1]


def _tc_final(accp_ref, y2_ref, dinv_ref, b_ref, batch_ref,
              wl1_ref, bl1_ref, wl2_ref, bl2_ref, out_ref):
    acc = accp_ref[0] + accp_ref[1] + y2_ref[...]
    h = jnp.maximum(acc * dinv_ref[:, 0:1] + b_ref[...], 0.0)
    seg = lax.broadcasted_iota(jnp.int32, (_G, _N), 0)
    p = (batch_ref[...] == seg).astype(jnp.float32)
    g = jnp.dot(p, h, preferred_element_type=jnp.float32)
    g1 = jnp.maximum(
        jnp.dot(g, wl1_ref[...], preferred_element_type=jnp.float32)
        + bl1_ref[...], 0.0)
    out_ref[...] = (
        jnp.dot(g1, wl2_ref[...], preferred_element_type=jnp.float32)
        + bl2_ref[...])


def kernel(x, edge_index, batch, Wc1, bc1, Wc2, bc2, Wl1, bl1, Wl2, bl2):
    src2d = edge_index[0].reshape(_R, _K)
    dst2d = edge_index[1].reshape(_R, _K)
    zeros_d = jnp.zeros((125, _DW), jnp.float32)
    ones_d = jnp.ones((_K, _DW), jnp.float32)
    zeros_h = jnp.zeros((125, _H), jnp.float32)

    degp = _deg(dst2d, zeros_d, ones_d)

    y1, dinv = pl.pallas_call(
        _tc_scale,
        out_shape=(
            jax.ShapeDtypeStruct((_N, _H), jnp.float32),
            jax.ShapeDtypeStruct((_N, _DW), jnp.float32),
        ),
    )(x, Wc1, degp)

    accp1 = _agg(y1, src2d, dst2d, zeros_h)

    y2 = pl.pallas_call(
        _tc_mid,
        out_shape=jax.ShapeDtypeStruct((_N, _H), jnp.float32),
    )(accp1, y1, dinv, bc1.reshape(1, _H), Wc2)

    accp2 = _agg(y2, src2d, dst2d, zeros_h)

    out = pl.pallas_call(
        _tc_final,
        out_shape=jax.ShapeDtypeStruct((_G, 10), jnp.float32),
    )(accp2, y2, dinv, bc2.reshape(1, _H), batch.reshape(1, _N),
      Wl1, bl1.reshape(1, _H), Wl2, bl2.reshape(1, 10))

    return out


# speedup vs baseline: 18.5558x; 18.5558x over previous
import functools

import jax
import jax.numpy as jnp
from jax import lax
from jax.experimental import pallas as pl
from jax.experimental.pallas import tpu as pltpu
from jax.experimental.pallas import tpu_sc as plsc

_N = 10000
_NP = 10240
_E = 320000
_H = 128
_G = 64

_K = 80
_NW = 32
_RW = _E // (_NW * _K)
_TNP = _NP // 16
_WB = 80

_sc_mesh = plsc.VectorSubcoreMesh(core_axis_name="c", subcore_axis_name="s")


def _make_agg(width):
    @functools.partial(
        pl.kernel,
        out_type=jax.ShapeDtypeStruct((2, _NP, width), jnp.float32),
        mesh=_sc_mesh,
        scratch_types=[
            pltpu.VMEM((_RW, _K), jnp.int32),
            pltpu.VMEM((_RW, _K), jnp.int32),
            pltpu.VMEM((_K, width), jnp.float32),
            pltpu.VMEM_SHARED((_NP, width), jnp.float32),
            pltpu.SemaphoreType.DMA,
        ],
    )
    def _agg_k(y, src3d, dst3d, zeros_hbm, out, sidx, didx, buf, acc, sem):
        c = lax.axis_index("c")
        s = lax.axis_index("s")
        wid = s * 2 + c
        pltpu.sync_copy(zeros_hbm, buf)
        for i in range(8):
            pltpu.sync_copy(buf, acc.at[pl.ds(s * _TNP + i * _WB, _WB)])
        pltpu.sync_copy(src3d.at[wid], sidx)
        pltpu.sync_copy(dst3d.at[wid], didx)
        plsc.subcore_barrier()

        def body(j, carry):
            pltpu.async_copy(y.at[sidx.at[j]], buf, sem).wait()
            pltpu.sync_copy(buf, acc.at[didx.at[j]], add=True)
            return carry

        lax.fori_loop(0, _RW, body, 0)
        plsc.subcore_barrier()
        for i in range(8):
            pltpu.sync_copy(acc.at[pl.ds(s * _TNP + i * _WB, _WB)], buf)
            pltpu.sync_copy(buf, out.at[c, pl.ds(s * _TNP + i * _WB, _WB)])

    return _agg_k


_agg = _make_agg(_H)


@functools.partial(
    pl.kernel,
    out_type=jax.ShapeDtypeStruct((2, _NP, _H), jnp.float32),
    mesh=_sc_mesh,
    scratch_types=[
        pltpu.VMEM((_RW, _K), jnp.int32),
        pltpu.VMEM((_K, _H), jnp.float32),
        pltpu.VMEM_SHARED((_NP, _H), jnp.float32),
    ],
)
def _deg(dst3d, zeros_hbm, ones_hbm, out, didx, buf, acc):
    c = lax.axis_index("c")
    s = lax.axis_index("s")
    wid = s * 2 + c
    pltpu.sync_copy(zeros_hbm, buf)
    for i in range(8):
        pltpu.sync_copy(buf, acc.at[pl.ds(s * _TNP + i * _WB, _WB)])
    pltpu.sync_copy(dst3d.at[wid], didx)
    pltpu.sync_copy(ones_hbm, buf)
    plsc.subcore_barrier()

    def body(j, carry):
        pltpu.sync_copy(buf, acc.at[didx.at[j]], add=True)
        return carry

    lax.fori_loop(0, _RW, body, 0)
    plsc.subcore_barrier()
    for i in range(8):
        pltpu.sync_copy(acc.at[pl.ds(s * _TNP + i * _WB, _WB)], buf)
        pltpu.sync_copy(buf, out.at[c, pl.ds(s * _TNP + i * _WB, _WB)])


def _tc_scale(x_ref, w_ref, degp_ref, y_ref, dinv_ref):
    deg = degp_ref[0, : _N] + degp_ref[1, : _N] + 1.0
    dinv = lax.rsqrt(jnp.maximum(deg, 1.0))
    xw = jnp.dot(x_ref[...], w_ref[...], preferred_element_type=jnp.float32)
    y_ref[...] = xw * dinv
    dinv_ref[...] = dinv


def _tc_mid(accp_ref, y_ref, dinv_ref, b_ref, w_ref, y2_ref):
    acc = accp_ref[0, : _N] + accp_ref[1, : _N] + y_ref[...]
    h = jnp.maximum(acc * dinv_ref[...] + b_ref[...], 0.0)
    xw = jnp.dot(h, w_ref[...], preferred_element_type=jnp.float32)
    y2_ref[...] = xw * dinv_ref[...]


def _tc_final(accp_ref, y2_ref, dinv_ref, b_ref, batch_ref,
              wl1_ref, bl1_ref, wl2_ref, bl2_ref, out_ref):
    acc = accp_ref[0, : _N] + accp_ref[1, : _N] + y2_ref[...]
    h = jnp.maximum(acc * dinv_ref[...] + b_ref[...], 0.0)
    seg = lax.broadcasted_iota(jnp.int32, (_G, _N), 0)
    p = (batch_ref[...] == seg).astype(jnp.float32)
    g = jnp.dot(p, h, preferred_element_type=jnp.float32)
    g1 = jnp.maximum(
        jnp.dot(g, wl1_ref[...], preferred_element_type=jnp.float32)
        + bl1_ref[...], 0.0)
    out_ref[...] = (
        jnp.dot(g1, wl2_ref[...], preferred_element_type=jnp.float32)
        + bl2_ref[...])


def kernel(x, edge_index, batch, Wc1, bc1, Wc2, bc2, Wl1, bl1, Wl2, bl2):
    src3d = edge_index[0].reshape(_NW, _RW, _K)
    dst3d = edge_index[1].reshape(_NW, _RW, _K)
    ones_h = jnp.ones((_K, _H), jnp.float32)
    zeros_h = jnp.zeros((_WB, _H), jnp.float32)

    degp = _deg(dst3d, zeros_h, ones_h)

    y1, dinv = pl.pallas_call(
        _tc_scale,
        out_shape=(
            jax.ShapeDtypeStruct((_N, _H), jnp.float32),
            jax.ShapeDtypeStruct((_N, _H), jnp.float32),
        ),
    )(x, Wc1, degp)

    accp1 = _agg(y1, src3d, dst3d, zeros_h)

    y2 = pl.pallas_call(
        _tc_mid,
        out_shape=jax.ShapeDtypeStruct((_N, _H), jnp.float32),
    )(accp1, y1, dinv, bc1.reshape(1, _H), Wc2)

    accp2 = _agg(y2, src3d, dst3d, zeros_h)

    out = pl.pallas_call(
        _tc_final,
        out_shape=jax.ShapeDtypeStruct((_G, 10), jnp.float32),
    )(accp2, y2, dinv, bc2.reshape(1, _H), batch.reshape(1, _N),
      Wl1, bl1.reshape(1, _H), Wl2, bl2.reshape(1, 10))

    return out
